# Initial kernel scaffold; baseline (speedup 1.0000x reference)
import jax
import jax.numpy as jnp
from jax.experimental import pallas as pl


def kernel(distance_bin_ids, embedding):
    raise NotImplementedError("write your pallas kernel here")


# speedup vs baseline: 2.9645x; 2.9645x over previous
import functools

import jax
import jax.numpy as jnp
from jax import lax
from jax.experimental import pallas as pl
from jax.experimental.pallas import tpu as pltpu
from jax.experimental.pallas import tpu_sc as plsc

NUM_ROWS = 129
DIM = 128
B = 4096
U = 200
TOTAL = B * U

NC = 2
NS = 16
NW = NC * NS
PER_W = TOTAL // NW
G = 128
KPC = 4
CHUNK = G * KPC
NCHUNK = PER_W // CHUNK
IDX_ROWS_PER_W = PER_W // G


def _sc_gather(idx2d, table):
    mesh = plsc.VectorSubcoreMesh(core_axis_name="c", subcore_axis_name="s")

    @functools.partial(
        pl.kernel,
        mesh=mesh,
        out_type=jax.ShapeDtypeStruct((TOTAL, DIM), jnp.float32),
        scratch_types=[
            pltpu.VMEM((KPC, G), jnp.int32),
            pltpu.VMEM((CHUNK, DIM), jnp.float32),
            pltpu.SemaphoreType.DMA,
        ],
    )
    def k(idx_hbm, table_hbm, out_hbm, idx_v, rows_v, sem):
        wid = lax.axis_index("s") * NC + lax.axis_index("c")
        row0 = wid * IDX_ROWS_PER_W

        def body(j, carry):
            pltpu.sync_copy(idx_hbm.at[pl.ds(row0 + j * KPC, KPC)], idx_v)
            cps = [
                pltpu.async_copy(
                    table_hbm.at[idx_v.at[t]],
                    rows_v.at[pl.ds(t * G, G)],
                    sem,
                )
                for t in range(KPC)
            ]
            for cp in cps:
                cp.wait()
            pltpu.sync_copy(
                rows_v, out_hbm.at[pl.ds((row0 + j * KPC) * G, CHUNK)]
            )
            return carry

        lax.fori_loop(0, NCHUNK, body, 0)

    return k(idx2d, table)


def kernel(distance_bin_ids, embedding):
    idx2d = distance_bin_ids.reshape(TOTAL // G, G).astype(jnp.int32)
    out = _sc_gather(idx2d, embedding)
    return out.reshape(B, U, DIM)


# speedup vs baseline: 2.9890x; 1.0083x over previous
import functools

import jax
import jax.numpy as jnp
from jax import lax
from jax.experimental import pallas as pl
from jax.experimental.pallas import tpu as pltpu
from jax.experimental.pallas import tpu_sc as plsc

NUM_ROWS = 129
DIM = 128
B = 4096
U = 200
TOTAL = B * U

NC = 2
NS = 16
NW = NC * NS
PER_W = TOTAL // NW
G = 128
KPC = 2
CHUNK = G * KPC
NBUF = 2
NCHUNK = PER_W // CHUNK
NOUTER = NCHUNK // NBUF
IDX_ROWS_PER_W = PER_W // G


def _sc_gather(idx2d, table):
    mesh = plsc.VectorSubcoreMesh(core_axis_name="c", subcore_axis_name="s")

    @functools.partial(
        pl.kernel,
        mesh=mesh,
        out_type=jax.ShapeDtypeStruct((TOTAL, DIM), jnp.float32),
        scratch_types=[
            pltpu.VMEM((IDX_ROWS_PER_W, G), jnp.int32),
            pltpu.VMEM((CHUNK, DIM), jnp.float32),
            pltpu.VMEM((CHUNK, DIM), jnp.float32),
            pltpu.SemaphoreType.DMA,
            pltpu.SemaphoreType.DMA,
            pltpu.SemaphoreType.DMA,
        ],
    )
    def k(idx_hbm, table_hbm, out_hbm, idx_v, rows_v0, rows_v1, gsem,
          ssem0, ssem1):
        wid = lax.axis_index("s") * NC + lax.axis_index("c")
        row0 = wid * IDX_ROWS_PER_W
        out0 = wid * PER_W
        rows_bufs = (rows_v0, rows_v1)
        ssems = (ssem0, ssem1)

        pltpu.sync_copy(idx_hbm.at[pl.ds(row0, IDX_ROWS_PER_W)], idx_v)

        def body(j, carry):
            for b in range(NBUF):
                rows_v = rows_bufs[b]
                ssem = ssems[b]
                c = j * NBUF + b

                @pl.when(j > 0)
                def _drain():
                    pltpu.make_async_copy(
                        rows_v, out_hbm.at[pl.ds(out0, CHUNK)], ssem
                    ).wait()

                cps = [
                    pltpu.async_copy(
                        table_hbm.at[idx_v.at[c * KPC + t]],
                        rows_v.at[pl.ds(t * G, G)],
                        gsem,
                    )
                    for t in range(KPC)
                ]
                for cp in cps:
                    cp.wait()

                pltpu.async_copy(
                    rows_v, out_hbm.at[pl.ds(out0 + c * CHUNK, CHUNK)], ssem
                )
            return carry

        lax.fori_loop(0, NOUTER, body, 0)

        for b in range(NBUF):
            pltpu.make_async_copy(
                rows_bufs[b], out_hbm.at[pl.ds(out0, CHUNK)], ssems[b]
            ).wait()

    return k(idx2d, table)


def kernel(distance_bin_ids, embedding):
    idx2d = distance_bin_ids.reshape(TOTAL // G, G).astype(jnp.int32)
    out = _sc_gather(idx2d, embedding)
    return out.reshape(B, U, DIM)


# speedup vs baseline: 15.4607x; 5.1725x over previous
import functools

import jax
import jax.numpy as jnp
from jax import lax
from jax.experimental import pallas as pl
from jax.experimental.pallas import tpu as pltpu
from jax.experimental.pallas import tpu_sc as plsc

NUM_ROWS = 129
DIM = 128
B = 4096
U = 200
TOTAL = B * U

NC = 2
NS = 16
NW = NC * NS
PER_W = TOTAL // NW
G = 128
KPC = 2
CHUNK = G * KPC
NBUF = 2
NCHUNK = PER_W // CHUNK
NOUTER = NCHUNK // NBUF
IDX_ROWS_PER_W = PER_W // G


def _sc_gather(idx2d, table):
    mesh = plsc.VectorSubcoreMesh(core_axis_name="c", subcore_axis_name="s")

    @functools.partial(
        pl.kernel,
        mesh=mesh,
        out_type=jax.ShapeDtypeStruct((TOTAL, DIM), jnp.float32),
        scratch_types=[
            pltpu.VMEM((IDX_ROWS_PER_W, G), jnp.int32),
            pltpu.VMEM((CHUNK, DIM), jnp.float32),
            pltpu.VMEM((CHUNK, DIM), jnp.float32),
            pltpu.VMEM_SHARED((NUM_ROWS, DIM), jnp.float32),
            pltpu.SemaphoreType.DMA,
            pltpu.SemaphoreType.DMA,
            pltpu.SemaphoreType.DMA,
        ],
    )
    def k(idx_hbm, table_hbm, out_hbm, idx_v, rows_v0, rows_v1, table_sh,
          gsem, ssem0, ssem1):
        sid = lax.axis_index("s")
        wid = sid * NC + lax.axis_index("c")
        row0 = wid * IDX_ROWS_PER_W
        out0 = wid * PER_W
        rows_bufs = (rows_v0, rows_v1)
        ssems = (ssem0, ssem1)

        @pl.when(sid == 0)
        def _stage_table():
            pltpu.sync_copy(table_hbm, table_sh)

        pltpu.sync_copy(idx_hbm.at[pl.ds(row0, IDX_ROWS_PER_W)], idx_v)
        plsc.subcore_barrier()

        def body(j, carry):
            for b in range(NBUF):
                rows_v = rows_bufs[b]
                ssem = ssems[b]
                c = j * NBUF + b

                @pl.when(j > 0)
                def _drain():
                    pltpu.make_async_copy(
                        rows_v, out_hbm.at[pl.ds(out0, CHUNK)], ssem
                    ).wait()

                cps = [
                    pltpu.async_copy(
                        table_sh.at[idx_v.at[c * KPC + t]],
                        rows_v.at[pl.ds(t * G, G)],
                        gsem,
                    )
                    for t in range(KPC)
                ]
                for cp in cps:
                    cp.wait()

                pltpu.async_copy(
                    rows_v, out_hbm.at[pl.ds(out0 + c * CHUNK, CHUNK)], ssem
                )
            return carry

        lax.fori_loop(0, NOUTER, body, 0)

        for b in range(NBUF):
            pltpu.make_async_copy(
                rows_bufs[b], out_hbm.at[pl.ds(out0, CHUNK)], ssems[b]
            ).wait()

    return k(idx2d, table)


def kernel(distance_bin_ids, embedding):
    idx2d = distance_bin_ids.reshape(TOTAL // G, G).astype(jnp.int32)
    out = _sc_gather(idx2d, embedding)
    return out.reshape(B, U, DIM)
